# Initial kernel scaffold; baseline (speedup 1.0000x reference)
import jax
import jax.numpy as jnp
from jax.experimental import pallas as pl


def kernel(font_output_data, font_target_data):
    raise NotImplementedError("write your pallas kernel here")


# speedup vs baseline: 1.2540x; 1.2540x over previous
import jax
import jax.numpy as jnp
from jax.experimental import pallas as pl
from jax.experimental.pallas import tpu as pltpu

_N_ROWS = 16384
_N_COLS = 512
_BLK = 1024
_GRID = _N_ROWS // _BLK
_TOTAL = float(_N_ROWS * _N_COLS)


def _loss_body(x_ref, t_ref, out_ref, acc_ref):
    i = pl.program_id(0)

    @pl.when(i == 0)
    def _init():
        acc_ref[0] = 0.0
        acc_ref[1] = 0.0
        acc_ref[2] = 0.0

    x = x_ref[...]
    t = t_ref[...]
    p = jax.nn.sigmoid(x)
    logp = jnp.maximum(jnp.log(p), -100.0)
    log1mp = jnp.maximum(jnp.log1p(-p), -100.0)
    bce = -(t * logp + (1.0 - t) * log1mp)
    acc_ref[0] += jnp.sum(bce * t)
    acc_ref[1] += jnp.sum(bce)
    acc_ref[2] += jnp.sum(t)

    @pl.when(i == _GRID - 1)
    def _finalize():
        pos_sum = acc_ref[0]
        all_sum = acc_ref[1]
        pos_cnt = acc_ref[2]
        neg_sum = all_sum - pos_sum
        pos_loss = 0.5 * pos_sum / jnp.maximum(pos_cnt, 1.0)
        neg_loss = 0.5 * neg_sum / jnp.maximum(_TOTAL - pos_cnt, 1.0)
        out_ref[0] = pos_loss + neg_loss
        out_ref[1] = pos_loss
        out_ref[2] = neg_loss


def kernel(font_output_data, font_target_data):
    out = pl.pallas_call(
        _loss_body,
        grid=(_GRID,),
        in_specs=[
            pl.BlockSpec((_BLK, _N_COLS), lambda i: (i, 0)),
            pl.BlockSpec((_BLK, _N_COLS), lambda i: (i, 0)),
        ],
        out_specs=pl.BlockSpec(memory_space=pltpu.SMEM),
        out_shape=jax.ShapeDtypeStruct((3,), jnp.float32),
        scratch_shapes=[pltpu.SMEM((3,), jnp.float32)],
    )(font_output_data, font_target_data)
    return (out[0], out[1], out[2])


# speedup vs baseline: 1.5377x; 1.2262x over previous
import jax
import jax.numpy as jnp
from jax.experimental import pallas as pl
from jax.experimental.pallas import tpu as pltpu

_N_ROWS = 16384
_N_COLS = 512
_BLK = 1024
_GRID = _N_ROWS // _BLK
_TOTAL = float(_N_ROWS * _N_COLS)


def _loss_body(x_ref, t_ref, out_ref, acc_ref):
    i = pl.program_id(0)

    @pl.when(i == 0)
    def _init():
        acc_ref[0] = 0.0
        acc_ref[1] = 0.0
        acc_ref[2] = 0.0

    x = x_ref[...]
    t = t_ref[...]
    y = x * (1.0 - 2.0 * t)
    bce = jnp.maximum(y, 0.0) + jnp.log1p(jnp.exp(-jnp.abs(y)))
    acc_ref[0] += jnp.sum(bce * t)
    acc_ref[1] += jnp.sum(bce)
    acc_ref[2] += jnp.sum(t)

    @pl.when(i == _GRID - 1)
    def _finalize():
        pos_sum = acc_ref[0]
        all_sum = acc_ref[1]
        pos_cnt = acc_ref[2]
        neg_sum = all_sum - pos_sum
        pos_loss = 0.5 * pos_sum / jnp.maximum(pos_cnt, 1.0)
        neg_loss = 0.5 * neg_sum / jnp.maximum(_TOTAL - pos_cnt, 1.0)
        out_ref[0] = pos_loss + neg_loss
        out_ref[1] = pos_loss
        out_ref[2] = neg_loss


def kernel(font_output_data, font_target_data):
    out = pl.pallas_call(
        _loss_body,
        grid=(_GRID,),
        in_specs=[
            pl.BlockSpec((_BLK, _N_COLS), lambda i: (i, 0)),
            pl.BlockSpec((_BLK, _N_COLS), lambda i: (i, 0)),
        ],
        out_specs=pl.BlockSpec(memory_space=pltpu.SMEM),
        out_shape=jax.ShapeDtypeStruct((3,), jnp.float32),
        scratch_shapes=[pltpu.SMEM((3,), jnp.float32)],
    )(font_output_data, font_target_data)
    return (out[0], out[1], out[2])


# speedup vs baseline: 1.8657x; 1.2134x over previous
import jax
import jax.numpy as jnp
from jax.experimental import pallas as pl
from jax.experimental.pallas import tpu as pltpu

_N_ROWS = 16384
_N_COLS = 512
_BLK = 1024
_GRID = _N_ROWS // _BLK
_TOTAL = float(_N_ROWS * _N_COLS)


def _loss_body(x_ref, t_ref, out_ref, acc_ref):
    i = pl.program_id(0)

    @pl.when(i == 0)
    def _init():
        acc_ref[0] = 0.0
        acc_ref[1] = 0.0
        acc_ref[2] = 0.0

    x = x_ref[...]
    t = t_ref[...]
    y = x * (1.0 - 2.0 * t)
    bce = jnp.maximum(y, 0.0) + jnp.log(1.0 + jnp.exp(-jnp.abs(y)))
    acc_ref[0] += jnp.sum(bce * t)
    acc_ref[1] += jnp.sum(bce)
    acc_ref[2] += jnp.sum(t)

    @pl.when(i == _GRID - 1)
    def _finalize():
        pos_sum = acc_ref[0]
        all_sum = acc_ref[1]
        pos_cnt = acc_ref[2]
        neg_sum = all_sum - pos_sum
        pos_loss = 0.5 * pos_sum / jnp.maximum(pos_cnt, 1.0)
        neg_loss = 0.5 * neg_sum / jnp.maximum(_TOTAL - pos_cnt, 1.0)
        out_ref[0] = pos_loss + neg_loss
        out_ref[1] = pos_loss
        out_ref[2] = neg_loss


def kernel(font_output_data, font_target_data):
    out = pl.pallas_call(
        _loss_body,
        grid=(_GRID,),
        in_specs=[
            pl.BlockSpec((_BLK, _N_COLS), lambda i: (i, 0)),
            pl.BlockSpec((_BLK, _N_COLS), lambda i: (i, 0)),
        ],
        out_specs=pl.BlockSpec(memory_space=pltpu.SMEM),
        out_shape=jax.ShapeDtypeStruct((3,), jnp.float32),
        scratch_shapes=[pltpu.SMEM((3,), jnp.float32)],
    )(font_output_data, font_target_data)
    return (out[0], out[1], out[2])
